# baseline (speedup 1.0000x reference)
import functools
import math

import jax
import jax.numpy as jnp
from jax import lax
from jax.experimental import pallas as pl
from jax.experimental.pallas import tpu as pltpu
from jax.experimental.pallas import tpu_sc as plsc

N_NODES = 10000
N_EDGES = 640000
STALK = 4
NPAD = 10240
FLATP = NPAD * STALK

NC = 2
NS = 16
NW = NC * NS
EDGES_PER_TILE = N_EDGES // NW
ECH = 4000
NCH = EDGES_PER_TILE // ECH
EBUF = 4096
EGROUPS = ECH // 16
HF = FLATP // 2
RCH = HF // NS


def _stalk_body(x_ref, w_ref, b_ref, o_ref):
    xst = lax.dot_general(w_ref[...], x_ref[...], (((1,), (1,)), ((), ())),
                          preferred_element_type=jnp.float32)
    o_ref[:, :N_NODES] = xst + b_ref[...]


def _edge_body(xs_hbm, src_hbm, dst_hbm, out0_hbm, out1_hbm,
               xs_v, acc_v, src0_v, src1_v, dst0_v, dst1_v, shared,
               sem_xs, sem_s, sem_d):
    src_b = (src0_v, src1_v)
    dst_b = (dst0_v, dst1_v)
    cid = lax.axis_index("c")
    sid = lax.axis_index("s")
    wid = sid * NC + cid
    base = wid * EDGES_PER_TILE

    cp_xs = pltpu.async_copy(xs_hbm, xs_v, sem_xs)
    cp_s = pltpu.async_copy(src_hbm.at[pl.ds(base, ECH)],
                            src0_v.at[pl.ds(0, ECH)], sem_s)
    cp_d = pltpu.async_copy(dst_hbm.at[pl.ds(base, ECH)],
                            dst0_v.at[pl.ds(0, ECH)], sem_d)

    zero = jnp.zeros((16,), jnp.float32)

    @plsc.parallel_loop(0, FLATP // 16, unroll=8)
    def _(i):
        acc_v[pl.ds(i * 16, 16)] = zero

    cp_xs.wait()

    for k in range(NCH):
        b = k % 2
        if k + 1 < NCH:
            nb = (k + 1) % 2
            off = base + (k + 1) * ECH
            cpn_s = pltpu.async_copy(src_hbm.at[pl.ds(off, ECH)],
                                     src_b[nb].at[pl.ds(0, ECH)], sem_s)
            cpn_d = pltpu.async_copy(dst_hbm.at[pl.ds(off, ECH)],
                                     dst_b[nb].at[pl.ds(0, ECH)], sem_d)
        cp_s.wait()
        cp_d.wait()
        sv, dv = src_b[b], dst_b[b]

        @plsc.parallel_loop(0, EGROUPS, unroll=10)
        def _(g):
            s16 = plsc.bitcast(sv[pl.ds(g * 16, 16)], jnp.int32)
            d16 = plsc.bitcast(dv[pl.ds(g * 16, 16)], jnp.int32)
            for c in range(STALK):
                si = s16 if c == 0 else s16 + (c * NPAD)
                di = d16 if c == 0 else d16 + (c * NPAD)
                gs = plsc.load_gather(xs_v, [si])
                gd = plsc.load_gather(xs_v, [di])
                df = gd - gs
                plsc.addupdate_scatter(acc_v, [di], df)
                plsc.addupdate_scatter(acc_v, [si], -df)

        if k + 1 < NCH:
            cp_s, cp_d = cpn_s, cpn_d

    for h in range(2):
        if h == 1:
            plsc.subcore_barrier()
        pltpu.sync_copy(acc_v.at[pl.ds(h * HF, HF)], shared.at[sid])
        plsc.subcore_barrier()

        col = sid * RCH
        cps = []
        for r in range(NS):
            cps.append(pltpu.async_copy(
                shared.at[r, pl.ds(col, RCH)],
                acc_v.at[pl.ds(h * HF + r * RCH, RCH)], sem_xs))
        for cp in cps:
            cp.wait()

        @plsc.parallel_loop(0, RCH // 16, unroll=4)
        def _(j):
            v = acc_v[pl.ds(h * HF + j * 16, 16)]
            for r in range(1, NS):
                v = v + acc_v[pl.ds(h * HF + r * RCH + j * 16, 16)]
            acc_v[pl.ds(h * HF + j * 16, 16)] = v

        off = h * HF + col

        @pl.when(cid == 0)
        def _():
            pltpu.sync_copy(acc_v.at[pl.ds(h * HF, RCH)],
                            out0_hbm.at[pl.ds(off, RCH)])

        @pl.when(cid == 1)
        def _():
            pltpu.sync_copy(acc_v.at[pl.ds(h * HF, RCH)],
                            out1_hbm.at[pl.ds(off, RCH)])


@functools.cache
def _make_edge_kernel():
    mesh = plsc.VectorSubcoreMesh(core_axis_name="c", subcore_axis_name="s")
    return pl.kernel(
        _edge_body,
        mesh=mesh,
        compiler_params=pltpu.CompilerParams(needs_layout_passes=False),
        out_type=(jax.ShapeDtypeStruct((FLATP,), jnp.float32),
                  jax.ShapeDtypeStruct((FLATP,), jnp.float32)),
        scratch_types=[
            pltpu.VMEM((FLATP,), jnp.float32),
            pltpu.VMEM((FLATP,), jnp.float32),
            pltpu.VMEM((EBUF,), jnp.float32),
            pltpu.VMEM((EBUF,), jnp.float32),
            pltpu.VMEM((EBUF,), jnp.float32),
            pltpu.VMEM((EBUF,), jnp.float32),
            pltpu.VMEM_SHARED((NS, HF), jnp.float32),
            pltpu.SemaphoreType.DMA,
            pltpu.SemaphoreType.DMA,
            pltpu.SemaphoreType.DMA,
        ],
    )


def _out_body(p0_ref, p1_ref, w_ref, b_ref, o_ref):
    agg = (p0_ref[...] + p1_ref[...])[:, :N_NODES]
    out = lax.dot_general(agg, w_ref[...], (((0,), (1,)), ((), ())),
                          preferred_element_type=jnp.float32)
    out = out + b_ref[...]
    o_ref[...] = out * 0.5 * (1.0 + lax.erf(out * (1.0 / math.sqrt(2.0))))


def kernel(x, edge_index, W_r, b_r, W_o, b_o):
    out_ch = W_o.shape[0]
    xst = pl.pallas_call(
        _stalk_body,
        out_shape=jax.ShapeDtypeStruct((STALK, NPAD), jnp.float32),
    )(x, W_r, b_r.reshape(STALK, 1))

    xs_flat = xst.reshape(FLATP)
    src = lax.bitcast_convert_type(edge_index[0], jnp.float32)
    dst = lax.bitcast_convert_type(edge_index[1], jnp.float32)
    p0, p1 = _make_edge_kernel()(xs_flat, src, dst)

    out = pl.pallas_call(
        _out_body,
        out_shape=jax.ShapeDtypeStruct((N_NODES, out_ch), jnp.float32),
    )(p0.reshape(STALK, NPAD), p1.reshape(STALK, NPAD), W_o,
      b_o.reshape(1, out_ch))
    return out


# speedup vs baseline: 1.0023x; 1.0023x over previous
import functools
import math

import jax
import jax.numpy as jnp
from jax import lax
from jax.experimental import pallas as pl
from jax.experimental.pallas import tpu as pltpu
from jax.experimental.pallas import tpu_sc as plsc

N_NODES = 10000
N_EDGES = 640000
STALK = 4
NPAD = 10240
FLATP = NPAD * STALK

NC = 2
NS = 16
NW = NC * NS
EDGES_PER_TILE = N_EDGES // NW
ECH = 4000
NCH = EDGES_PER_TILE // ECH
EBUF = 4096
EGROUPS = ECH // 16
HF = FLATP // 2
RCH = HF // NS


def _stalk_body(x_ref, w_ref, b_ref, o_ref):
    xst = lax.dot_general(w_ref[...], x_ref[...], (((1,), (1,)), ((), ())),
                          preferred_element_type=jnp.float32)
    o_ref[:, :N_NODES] = xst + b_ref[...]


def _edge_body(xs_hbm, src_hbm, dst_hbm, out0_hbm, out1_hbm,
               xs_v, acc_v, src0_v, src1_v, dst0_v, dst1_v, shared,
               sem_xs, sem_s, sem_d):
    src_b = (src0_v, src1_v)
    dst_b = (dst0_v, dst1_v)
    cid = lax.axis_index("c")
    sid = lax.axis_index("s")
    wid = sid * NC + cid
    base = wid * EDGES_PER_TILE

    cp_xs = pltpu.async_copy(xs_hbm, xs_v, sem_xs)
    cp_s = pltpu.async_copy(src_hbm.at[pl.ds(base, ECH)],
                            src0_v.at[pl.ds(0, ECH)], sem_s)
    cp_d = pltpu.async_copy(dst_hbm.at[pl.ds(base, ECH)],
                            dst0_v.at[pl.ds(0, ECH)], sem_d)

    zero = jnp.zeros((16,), jnp.float32)

    @plsc.parallel_loop(0, FLATP // 16, unroll=8)
    def _(i):
        acc_v[pl.ds(i * 16, 16)] = zero

    cp_xs.wait()

    for k in range(NCH):
        b = k % 2
        if k + 1 < NCH:
            nb = (k + 1) % 2
            off = base + (k + 1) * ECH
            cpn_s = pltpu.async_copy(src_hbm.at[pl.ds(off, ECH)],
                                     src_b[nb].at[pl.ds(0, ECH)], sem_s)
            cpn_d = pltpu.async_copy(dst_hbm.at[pl.ds(off, ECH)],
                                     dst_b[nb].at[pl.ds(0, ECH)], sem_d)
        cp_s.wait()
        cp_d.wait()
        sv, dv = src_b[b], dst_b[b]

        @plsc.parallel_loop(0, EGROUPS, unroll=10)
        def _(g):
            s16 = plsc.bitcast(sv[pl.ds(g * 16, 16)], jnp.int32)
            d16 = plsc.bitcast(dv[pl.ds(g * 16, 16)], jnp.int32)
            for c in range(STALK):
                si = s16 if c == 0 else s16 + (c * NPAD)
                di = d16 if c == 0 else d16 + (c * NPAD)
                gs = plsc.load_gather(xs_v, [si])
                gd = plsc.load_gather(xs_v, [di])
                df = gd - gs
                plsc.addupdate_scatter(acc_v, [di], df)
                plsc.addupdate_scatter(acc_v, [si], -df)

        if k + 1 < NCH:
            cp_s, cp_d = cpn_s, cpn_d

    for h in range(2):
        if h == 1:
            plsc.subcore_barrier()
        pltpu.sync_copy(acc_v.at[pl.ds(h * HF, HF)], shared.at[sid])
        plsc.subcore_barrier()

        col = sid * RCH
        cps = []
        for r in range(NS):
            cps.append(pltpu.async_copy(
                shared.at[r, pl.ds(col, RCH)],
                acc_v.at[pl.ds(h * HF + r * RCH, RCH)], sem_xs))
        for cp in cps:
            cp.wait()

        @plsc.parallel_loop(0, RCH // 16, unroll=4)
        def _(j):
            v = acc_v[pl.ds(h * HF + j * 16, 16)]
            for r in range(1, NS):
                v = v + acc_v[pl.ds(h * HF + r * RCH + j * 16, 16)]
            acc_v[pl.ds(h * HF + j * 16, 16)] = v

        off = h * HF + col

        @pl.when(cid == 0)
        def _():
            pltpu.sync_copy(acc_v.at[pl.ds(h * HF, RCH)],
                            out0_hbm.at[pl.ds(off, RCH)])

        @pl.when(cid == 1)
        def _():
            pltpu.sync_copy(acc_v.at[pl.ds(h * HF, RCH)],
                            out1_hbm.at[pl.ds(off, RCH)])


@functools.cache
def _make_edge_kernel():
    mesh = plsc.VectorSubcoreMesh(core_axis_name="c", subcore_axis_name="s")
    return pl.kernel(
        _edge_body,
        mesh=mesh,
        compiler_params=pltpu.CompilerParams(needs_layout_passes=False,
                                             disable_bounds_checks=True),
        out_type=(jax.ShapeDtypeStruct((FLATP,), jnp.float32),
                  jax.ShapeDtypeStruct((FLATP,), jnp.float32)),
        scratch_types=[
            pltpu.VMEM((FLATP,), jnp.float32),
            pltpu.VMEM((FLATP,), jnp.float32),
            pltpu.VMEM((EBUF,), jnp.float32),
            pltpu.VMEM((EBUF,), jnp.float32),
            pltpu.VMEM((EBUF,), jnp.float32),
            pltpu.VMEM((EBUF,), jnp.float32),
            pltpu.VMEM_SHARED((NS, HF), jnp.float32),
            pltpu.SemaphoreType.DMA,
            pltpu.SemaphoreType.DMA,
            pltpu.SemaphoreType.DMA,
        ],
    )


def _out_body(p0_ref, p1_ref, w_ref, b_ref, o_ref):
    agg = (p0_ref[...] + p1_ref[...])[:, :N_NODES]
    out = lax.dot_general(agg, w_ref[...], (((0,), (1,)), ((), ())),
                          preferred_element_type=jnp.float32)
    out = out + b_ref[...]
    o_ref[...] = out * 0.5 * (1.0 + lax.erf(out * (1.0 / math.sqrt(2.0))))


def kernel(x, edge_index, W_r, b_r, W_o, b_o):
    out_ch = W_o.shape[0]
    xst = pl.pallas_call(
        _stalk_body,
        out_shape=jax.ShapeDtypeStruct((STALK, NPAD), jnp.float32),
    )(x, W_r, b_r.reshape(STALK, 1))

    xs_flat = xst.reshape(FLATP)
    src = lax.bitcast_convert_type(edge_index[0], jnp.float32)
    dst = lax.bitcast_convert_type(edge_index[1], jnp.float32)
    p0, p1 = _make_edge_kernel()(xs_flat, src, dst)

    out = pl.pallas_call(
        _out_body,
        out_shape=jax.ShapeDtypeStruct((N_NODES, out_ch), jnp.float32),
    )(p0.reshape(STALK, NPAD), p1.reshape(STALK, NPAD), W_o,
      b_o.reshape(1, out_ch))
    return out
